# baseline (speedup 1.0000x reference)
import functools

import jax
import jax.numpy as jnp
import numpy as np
from jax import lax
from jax.experimental import pallas as pl
from jax.experimental.pallas import tpu as pltpu

_NEG = np.float32(-1e9)


def _fused_body(k_ref, s_ref, l_ref, o_ref, scr, cs, ls, keep):
    j = pl.program_id(0)

    @pl.when(j < 4)
    def _stats():
        s = s_ref[...]
        scr[:, pl.ds(j * 8192, 8192)] = s
        colsum = jnp.sum(jax.nn.sigmoid(s), axis=0)
        lsum = jnp.sum(l_ref[...], axis=0)
        for t in range(64):
            row = pl.ds(j * 64 + t, 1)
            cs[row, :] = colsum[t * 128:(t + 1) * 128].reshape(1, 128)
            ls[row, :] = lsum[t * 128:(t + 1) * 128].reshape(1, 128)

    @pl.when(j == 4)
    def _select():
        v = cs[...]
        key = lax.bitcast_convert_type(v, jnp.int32)
        k = k_ref[0]

        def grp_step(i, t):
            shift = jnp.int32(27) - 4 * i
            bits4 = jnp.int32(0)
            for j in range(1, 16):
                cand = t | (jnp.int32(j) << shift)
                cnt = jnp.sum((key >= cand).astype(jnp.int32))
                bits4 = bits4 + (cnt >= k).astype(jnp.int32)
            return t | (bits4 << shift)

        t_final = lax.fori_loop(0, 7, grp_step, jnp.int32(0), unroll=True)
        bits3 = jnp.int32(0)
        for j in range(1, 8):
            cand = t_final | jnp.int32(j)
            cnt = jnp.sum((key >= cand).astype(jnp.int32))
            bits3 = bits3 + (cnt >= k).astype(jnp.int32)
        t_final = t_final | bits3

        count_gt = jnp.sum((key > t_final).astype(jnp.int32))
        r = (k - count_gt).astype(jnp.float32)

        eq = key == t_final
        ef = eq.astype(jnp.float32)
        li = lax.broadcasted_iota(jnp.int32, (128, 128), 0)
        lj = lax.broadcasted_iota(jnp.int32, (128, 128), 1)
        lt_strict = (li < lj).astype(jnp.float32)
        ri = lax.broadcasted_iota(jnp.int32, (256, 256), 0)
        rj = lax.broadcasted_iota(jnp.int32, (256, 256), 1)
        rt_strict = (ri > rj).astype(jnp.float32)
        ones = jnp.ones((128, 128), jnp.float32)

        pref_row = jnp.dot(ef, lt_strict, preferred_element_type=jnp.float32)
        row_tot = jnp.dot(ef, ones, preferred_element_type=jnp.float32)
        pref_rows = jnp.dot(rt_strict, row_tot,
                            preferred_element_type=jnp.float32)
        prefix = pref_row + pref_rows

        kp = (key > t_final) | (eq & (prefix < r)) | (ls[...] > 0)
        keep[...] = kp.astype(jnp.float32)

    @pl.when(j >= 5)
    def _mask():
        jj = j - 5
        kp = jnp.concatenate(
            [keep[pl.ds(jj * 64 + t, 1), :] for t in range(64)], axis=1)
        s = scr[:, pl.ds(jj * 8192, 8192)]
        o_ref[...] = jnp.where(kp > 0.0, s, _NEG)


@jax.jit
def kernel(scores, label, k):
    B, N = scores.shape
    CB = 8192
    nblk = N // CB
    k_arr = jnp.asarray(k, jnp.int32).reshape(1)

    out = pl.pallas_call(
        _fused_body,
        grid=(2 * nblk + 1,),
        in_specs=[
            pl.BlockSpec(memory_space=pltpu.SMEM),
            pl.BlockSpec((B, CB), lambda j: (0, jnp.minimum(j, 3))),
            pl.BlockSpec((B, CB), lambda j: (0, jnp.minimum(j, 3))),
        ],
        out_specs=pl.BlockSpec(
            (B, CB), lambda j: (0, jnp.maximum(j - 5, 0))),
        out_shape=jax.ShapeDtypeStruct((B, N), jnp.float32),
        scratch_shapes=[
            pltpu.VMEM((B, N), jnp.float32),
            pltpu.VMEM((256, 128), jnp.float32),
            pltpu.VMEM((256, 128), jnp.int32),
            pltpu.VMEM((256, 128), jnp.float32),
        ],
        compiler_params=pltpu.CompilerParams(
            dimension_semantics=("arbitrary",)),
    )(k_arr, scores, label)
    return out


# speedup vs baseline: 1.0076x; 1.0076x over previous
import functools

import jax
import jax.numpy as jnp
import numpy as np
from jax import lax
from jax.experimental import pallas as pl
from jax.experimental.pallas import tpu as pltpu

_NEG = np.float32(-1e9)


def _fused_body(k_ref, s_ref, l_ref, o_ref, scr, cs, ls, keep):
    j = pl.program_id(0)

    @pl.when(j < 4)
    def _stats():
        s = s_ref[...]
        scr[:, pl.ds(j * 8192, 8192)] = s
        colsum = jnp.sum(jax.nn.sigmoid(s), axis=0)
        lsum = jnp.sum(l_ref[...], axis=0)
        for t in range(64):
            row = pl.ds(j * 64 + t, 1)
            cs[row, :] = colsum[t * 128:(t + 1) * 128].reshape(1, 128)
            ls[row, :] = lsum[t * 128:(t + 1) * 128].reshape(1, 128)

    @pl.when(j == 4)
    def _select():
        v = cs[...]
        key = lax.bitcast_convert_type(v, jnp.int32)
        k = k_ref[0]

        def grp_step(i, t):
            shift = jnp.int32(28) - 3 * i
            bits3 = jnp.int32(0)
            for j in range(1, 8):
                cand = t | (jnp.int32(j) << shift)
                cnt = jnp.sum((key >= cand).astype(jnp.int32))
                bits3 = bits3 + (cnt >= k).astype(jnp.int32)
            return t | (bits3 << shift)

        t_final = lax.fori_loop(0, 10, grp_step, jnp.int32(0), unroll=True)
        cand0 = t_final | jnp.int32(1)
        cnt0 = jnp.sum((key >= cand0).astype(jnp.int32))
        t_final = jnp.where(cnt0 >= k, cand0, t_final)

        count_gt = jnp.sum((key > t_final).astype(jnp.int32))
        r = (k - count_gt).astype(jnp.float32)

        eq = key == t_final
        ef = eq.astype(jnp.float32)
        li = lax.broadcasted_iota(jnp.int32, (128, 128), 0)
        lj = lax.broadcasted_iota(jnp.int32, (128, 128), 1)
        lt_strict = (li < lj).astype(jnp.float32)
        ri = lax.broadcasted_iota(jnp.int32, (256, 256), 0)
        rj = lax.broadcasted_iota(jnp.int32, (256, 256), 1)
        rt_strict = (ri > rj).astype(jnp.float32)
        ones = jnp.ones((128, 128), jnp.float32)

        pref_row = jnp.dot(ef, lt_strict, preferred_element_type=jnp.float32)
        row_tot = jnp.dot(ef, ones, preferred_element_type=jnp.float32)
        pref_rows = jnp.dot(rt_strict, row_tot,
                            preferred_element_type=jnp.float32)
        prefix = pref_row + pref_rows

        kp = (key > t_final) | (eq & (prefix < r)) | (ls[...] > 0)
        keep[...] = kp.astype(jnp.float32)

    @pl.when(j >= 5)
    def _mask():
        jj = j - 5
        kp = jnp.concatenate(
            [keep[pl.ds(jj * 64 + t, 1), :] for t in range(64)], axis=1)
        s = scr[:, pl.ds(jj * 8192, 8192)]
        o_ref[...] = jnp.where(kp > 0.0, s, _NEG)


@jax.jit
def kernel(scores, label, k):
    B, N = scores.shape
    CB = 8192
    nblk = N // CB
    k_arr = jnp.asarray(k, jnp.int32).reshape(1)

    out = pl.pallas_call(
        _fused_body,
        grid=(2 * nblk + 1,),
        in_specs=[
            pl.BlockSpec(memory_space=pltpu.SMEM),
            pl.BlockSpec((B, CB), lambda j: (0, jnp.minimum(j, 3))),
            pl.BlockSpec((B, CB), lambda j: (0, jnp.minimum(j, 3))),
        ],
        out_specs=pl.BlockSpec(
            (B, CB), lambda j: (0, jnp.maximum(j - 5, 0))),
        out_shape=jax.ShapeDtypeStruct((B, N), jnp.float32),
        scratch_shapes=[
            pltpu.VMEM((B, N), jnp.float32),
            pltpu.VMEM((256, 128), jnp.float32),
            pltpu.VMEM((256, 128), jnp.int32),
            pltpu.VMEM((256, 128), jnp.float32),
        ],
        compiler_params=pltpu.CompilerParams(
            dimension_semantics=("arbitrary",)),
    )(k_arr, scores, label)
    return out
